# baseline (device time: 118309 ns/iter reference)
import jax
import jax.numpy as jnp
from jax import lax
from jax.experimental import pallas as pl
from jax.experimental.pallas import tpu as pltpu

B, S, H, Dh, Dr = 2, 512, 16, 128, 32
D = 2048
DC = 128
SCALE = (Dh + Dr) ** -0.5

_CompilerParams = getattr(pltpu, "CompilerParams", None) or pltpu.TPUCompilerParams


def kernel(x, Wdkv, Wuk, Wuv, Wq, Wqr, Wkr, Wo):
    def body(
        x_ref, Wdkv_ref, Wuk_ref, Wuv_ref, Wq_ref, Wqr_ref, Wkr_ref, Wo_ref,
        out_ref,
        xbf, kbuf, vbuf, qrbuf, krbuf,
        c_send, c_recv, wk_send, wk_recv, wv_send, wv_recv,
        wqc, woc,
        send_sems, recv_sems, copy_sems,
    ):
        my_x = lax.axis_index("x")
        my_y = lax.axis_index("y")
        my_z = lax.axis_index("z")
        peer = (my_x, 1 - my_y, my_z)

        barrier = pltpu.get_barrier_semaphore()
        pl.semaphore_signal(
            barrier, inc=1, device_id=peer, device_id_type=pl.DeviceIdType.MESH
        )
        pl.semaphore_wait(barrier, 1)

        wk_send[...] = Wuk_ref[...].astype(jnp.bfloat16)
        wv_send[...] = Wuv_ref[...].astype(jnp.bfloat16)
        rdma_wk = pltpu.make_async_remote_copy(
            src_ref=wk_send, dst_ref=wk_recv,
            send_sem=send_sems.at[0], recv_sem=recv_sems.at[0],
            device_id=peer, device_id_type=pl.DeviceIdType.MESH,
        )
        rdma_wk.start()
        rdma_wv = pltpu.make_async_remote_copy(
            src_ref=wv_send, dst_ref=wv_recv,
            send_sem=send_sems.at[1], recv_sem=recv_sems.at[1],
            device_id=peer, device_id_type=pl.DeviceIdType.MESH,
        )
        rdma_wv.start()

        xbf[...] = x_ref[...].astype(jnp.bfloat16)
        wdkv_bf = Wdkv_ref[...].astype(jnp.bfloat16)
        for b in range(B):
            c_send[b, :, :] = jnp.dot(
                xbf[b, :, :], wdkv_bf, preferred_element_type=jnp.float32
            ).astype(jnp.bfloat16)
        rdma_c = pltpu.make_async_remote_copy(
            src_ref=c_send, dst_ref=c_recv,
            send_sem=send_sems.at[2], recv_sem=recv_sems.at[2],
            device_id=peer, device_id_type=pl.DeviceIdType.MESH,
        )
        rdma_c.start()

        wqr_bf = Wqr_ref[...].astype(jnp.bfloat16)
        wkr_bf = Wkr_ref[...].astype(jnp.bfloat16)
        for b in range(B):
            qrbuf[b, :, :] = (jnp.dot(
                xbf[b, :, :], wqr_bf, preferred_element_type=jnp.float32
            ) * SCALE).astype(jnp.bfloat16)
            krbuf[b, :, :] = jnp.dot(
                xbf[b, :, :], wkr_bf, preferred_element_type=jnp.float32
            ).astype(jnp.bfloat16)
        for b in range(B):
            kbuf[b, :, :] = jnp.dot(
                c_send[b, :, :], wk_send[...], preferred_element_type=jnp.float32
            ).astype(jnp.bfloat16)
            vbuf[b, :, :] = jnp.dot(
                c_send[b, :, :], wv_send[...], preferred_element_type=jnp.float32
            ).astype(jnp.bfloat16)

        q_cps = []
        o_cps = []
        for h in range(H):
            q_cps.append(pltpu.make_async_copy(
                Wq_ref.at[:, h * Dh:(h + 1) * Dh], wqc.at[h % 2],
                copy_sems.at[h % 2],
            ))
            o_cps.append(pltpu.make_async_copy(
                Wo_ref.at[h * Dh:(h + 1) * Dh, :], woc.at[h % 2],
                copy_sems.at[2 + h % 2],
            ))
        q_cps[0].start()
        o_cps[0].start()
        q_cps[1].start()
        o_cps[1].start()

        rdma_wk.wait()
        rdma_wv.wait()
        rdma_c.wait()
        for b in range(B):
            kbuf[b, :, :] = (
                kbuf[b, :, :].astype(jnp.float32)
                + jnp.dot(
                    c_recv[b, :, :], wk_recv[...],
                    preferred_element_type=jnp.float32,
                )
            ).astype(jnp.bfloat16)
            vbuf[b, :, :] = (
                vbuf[b, :, :].astype(jnp.float32)
                + jnp.dot(
                    c_recv[b, :, :], wv_recv[...],
                    preferred_element_type=jnp.float32,
                )
            ).astype(jnp.bfloat16)

        nt = (((1,), (1,)), ((), ()))
        for h in range(H):
            q_cps[h].wait()
            o_cps[h].wait()
            wq_bf = wqc[h % 2, :, :].astype(jnp.bfloat16)
            wo_bf = woc[h % 2, :, :].astype(jnp.bfloat16)
            if h + 2 < H:
                q_cps[h + 2].start()
                o_cps[h + 2].start()
            q_hb = (jnp.dot(
                jnp.reshape(xbf[...], (B * S, D)), wq_bf,
                preferred_element_type=jnp.float32,
            ) * SCALE).astype(jnp.bfloat16)
            for b in range(B):
                q_h = q_hb[b * S:(b + 1) * S, :]
                k_h = kbuf[b, :, h * Dh:(h + 1) * Dh]
                v_h = vbuf[b, :, h * Dh:(h + 1) * Dh]
                qr_h = qrbuf[b, :, h * Dr:(h + 1) * Dr]
                s = lax.dot_general(
                    q_h, k_h, nt, preferred_element_type=jnp.float32
                )
                s = s + lax.dot_general(
                    qr_h, krbuf[b, :, :], nt, preferred_element_type=jnp.float32
                )
                p = jnp.exp(s.astype(jnp.bfloat16))
                r = jnp.sum(p, axis=1, keepdims=True, dtype=jnp.float32)
                o_h = jnp.dot(
                    p, v_h, preferred_element_type=jnp.float32
                )
                o_n = o_h * (1.0 / r)
                contrib = jnp.dot(
                    o_n.astype(jnp.bfloat16), wo_bf,
                    preferred_element_type=jnp.float32,
                )
                if h == 0:
                    out_ref[b, :, :] = contrib
                else:
                    out_ref[b, :, :] = out_ref[b, :, :] + contrib

    vmem = pl.BlockSpec(memory_space=pltpu.VMEM)
    hbm = pl.BlockSpec(memory_space=pl.ANY)
    return pl.pallas_call(
        body,
        out_shape=jax.ShapeDtypeStruct((B, S, D), jnp.float32),
        in_specs=[vmem, vmem, vmem, vmem, hbm, vmem, vmem, hbm],
        out_specs=vmem,
        scratch_shapes=[
            pltpu.VMEM((B, S, D), jnp.bfloat16),
            pltpu.VMEM((B, S, D), jnp.bfloat16),
            pltpu.VMEM((B, S, D), jnp.bfloat16),
            pltpu.VMEM((B, S, H * Dr), jnp.bfloat16),
            pltpu.VMEM((B, S, Dr), jnp.bfloat16),
            pltpu.VMEM((B, S, DC), jnp.bfloat16),
            pltpu.VMEM((B, S, DC), jnp.bfloat16),
            pltpu.VMEM((DC, D), jnp.bfloat16),
            pltpu.VMEM((DC, D), jnp.bfloat16),
            pltpu.VMEM((DC, D), jnp.bfloat16),
            pltpu.VMEM((DC, D), jnp.bfloat16),
            pltpu.VMEM((2, D, Dh), jnp.float32),
            pltpu.VMEM((2, Dh, D), jnp.float32),
            pltpu.SemaphoreType.DMA((3,)),
            pltpu.SemaphoreType.DMA((3,)),
            pltpu.SemaphoreType.DMA((4,)),
        ],
        compiler_params=_CompilerParams(collective_id=0),
    )(x, Wdkv, Wuk, Wuv, Wq, Wqr, Wkr, Wo)


# device time: 63402 ns/iter; 1.8660x vs baseline; 1.8660x over previous
import jax
import jax.numpy as jnp
from jax import lax
from jax.experimental import pallas as pl
from jax.experimental.pallas import tpu as pltpu

B, S, H, Dh, Dr = 2, 512, 16, 128, 32
D = 2048
DC = 128
CK = 256
SCALE = (Dh + Dr) ** -0.5

_CompilerParams = getattr(pltpu, "CompilerParams", None) or pltpu.TPUCompilerParams


def kernel(x, Wdkv, Wuk, Wuv, Wq, Wqr, Wkr, Wo):
    x_bf = x.astype(jnp.bfloat16)

    def body(
        x_ref, Wdkv_ref, Wuk_ref, Wuv_ref, Wq_ref, Wqr_ref, Wkr_ref, Wo_ref,
        out_ref,
        kbuf, vbuf, qrbuf, krbuf,
        c_send, c_recv, wk_send, wk_recv, wv_send, wv_recv,
        wqc, woc, wkrc,
        send_sems, recv_sems, copy_sems,
    ):
        my_x = lax.axis_index("x")
        my_y = lax.axis_index("y")
        my_z = lax.axis_index("z")
        peer = (my_x, 1 - my_y, my_z)

        barrier = pltpu.get_barrier_semaphore()
        pl.semaphore_signal(
            barrier, inc=1, device_id=peer, device_id_type=pl.DeviceIdType.MESH
        )
        pl.semaphore_wait(barrier, 1)

        kr_cp = pltpu.make_async_copy(Wkr_ref, wkrc, copy_sems.at[4])
        kr_cp.start()

        wk_send[...] = Wuk_ref[...].astype(jnp.bfloat16)
        wv_send[...] = Wuv_ref[...].astype(jnp.bfloat16)
        rdma_wk = pltpu.make_async_remote_copy(
            src_ref=wk_send, dst_ref=wk_recv,
            send_sem=send_sems.at[0], recv_sem=recv_sems.at[0],
            device_id=peer, device_id_type=pl.DeviceIdType.MESH,
        )
        rdma_wk.start()
        rdma_wv = pltpu.make_async_remote_copy(
            src_ref=wv_send, dst_ref=wv_recv,
            send_sem=send_sems.at[1], recv_sem=recv_sems.at[1],
            device_id=peer, device_id_type=pl.DeviceIdType.MESH,
        )
        rdma_wv.start()

        wdkv_bf = Wdkv_ref[...].astype(jnp.bfloat16)
        for b in range(B):
            c_send[b, :, :] = jnp.dot(
                x_ref[b, :, :], wdkv_bf, preferred_element_type=jnp.float32
            ).astype(jnp.bfloat16)
        rdma_c = pltpu.make_async_remote_copy(
            src_ref=c_send, dst_ref=c_recv,
            send_sem=send_sems.at[2], recv_sem=recv_sems.at[2],
            device_id=peer, device_id_type=pl.DeviceIdType.MESH,
        )
        rdma_c.start()

        q_cps = []
        for r in range(D // CK):
            q_cps.append(pltpu.make_async_copy(
                Wq_ref.at[r * CK:(r + 1) * CK, :], wqc.at[r % 2],
                copy_sems.at[r % 2],
            ))
        q_cps[0].start()
        q_cps[1].start()

        wqr_bf = Wqr_ref[...].astype(jnp.bfloat16)
        kr_cp.wait()
        wkr_bf = wkrc[...].astype(jnp.bfloat16)
        for b in range(B):
            qrbuf[b, :, :] = (jnp.dot(
                x_ref[b, :, :], wqr_bf, preferred_element_type=jnp.float32
            ) * SCALE).astype(jnp.bfloat16)
            krbuf[b, :, :] = jnp.dot(
                x_ref[b, :, :], wkr_bf, preferred_element_type=jnp.float32
            ).astype(jnp.bfloat16)
        for b in range(B):
            kbuf[b, :, :] = jnp.dot(
                c_send[b, :, :], wk_send[...], preferred_element_type=jnp.float32
            ).astype(jnp.bfloat16)

        for r in range(D // CK):
            q_cps[r].wait()
            wq_bf = (wqc[r % 2, :, :] * SCALE).astype(jnp.bfloat16)
            if r + 2 < D // CK:
                q_cps[r + 2].start()
            for b in range(B):
                acc = jnp.dot(
                    x_ref[b, :, r * CK:(r + 1) * CK], wq_bf,
                    preferred_element_type=jnp.float32,
                )
                if r == 0:
                    vbuf[b, :, :] = acc.astype(jnp.bfloat16)
                else:
                    vbuf[b, :, :] = (vbuf[b, :, :] + acc).astype(jnp.bfloat16)

        rdma_wk.wait()
        rdma_wv.wait()
        rdma_c.wait()
        for b in range(B):
            kbuf[b, :, :] = (
                kbuf[b, :, :].astype(jnp.float32)
                + jnp.dot(
                    c_recv[b, :, :], wk_recv[...],
                    preferred_element_type=jnp.float32,
                )
            ).astype(jnp.bfloat16)

        o_cps = []
        for ci in range(D // CK):
            o_cps.append(pltpu.make_async_copy(
                Wo_ref.at[ci * CK:(ci + 1) * CK, :], woc.at[ci % 2],
                copy_sems.at[2 + ci % 2],
            ))
        o_cps[0].start()
        o_cps[1].start()

        nt = (((1,), (1,)), ((), ()))
        for h in range(H):
            hs = slice(h * Dh, (h + 1) * Dh)
            for b in range(B):
                q_h = vbuf[b, :, hs]
                k_h = kbuf[b, :, hs]
                qr_h = qrbuf[b, :, h * Dr:(h + 1) * Dr]
                s = lax.dot_general(
                    q_h, k_h, nt, preferred_element_type=jnp.float32
                )
                s = s + lax.dot_general(
                    qr_h, krbuf[b, :, :], nt, preferred_element_type=jnp.float32
                )
                p = jnp.exp(s.astype(jnp.bfloat16))
                r = jnp.sum(p, axis=1, keepdims=True, dtype=jnp.float32)
                v_h = (
                    jnp.dot(
                        c_send[b, :, :], wv_send[:, hs],
                        preferred_element_type=jnp.float32,
                    )
                    + jnp.dot(
                        c_recv[b, :, :], wv_recv[:, hs],
                        preferred_element_type=jnp.float32,
                    )
                ).astype(jnp.bfloat16)
                o_h = jnp.dot(p, v_h, preferred_element_type=jnp.float32)
                o_n = o_h * (1.0 / r)
                kbuf[b, :, hs] = o_n.astype(jnp.bfloat16)

        for ci in range(D // CK):
            o_cps[ci].wait()
            wo_bf = woc[ci % 2, :, :].astype(jnp.bfloat16)
            if ci + 2 < D // CK:
                o_cps[ci + 2].start()
            for b in range(B):
                d = jnp.dot(
                    kbuf[b, :, ci * CK:(ci + 1) * CK], wo_bf,
                    preferred_element_type=jnp.float32,
                )
                if ci == 0:
                    out_ref[b, :, :] = d
                else:
                    out_ref[b, :, :] = out_ref[b, :, :] + d

    vmem = pl.BlockSpec(memory_space=pltpu.VMEM)
    hbm = pl.BlockSpec(memory_space=pl.ANY)
    return pl.pallas_call(
        body,
        out_shape=jax.ShapeDtypeStruct((B, S, D), jnp.float32),
        in_specs=[vmem, vmem, vmem, vmem, hbm, vmem, hbm, hbm],
        out_specs=vmem,
        scratch_shapes=[
            pltpu.VMEM((B, S, D), jnp.bfloat16),
            pltpu.VMEM((B, S, D), jnp.bfloat16),
            pltpu.VMEM((B, S, H * Dr), jnp.bfloat16),
            pltpu.VMEM((B, S, Dr), jnp.bfloat16),
            pltpu.VMEM((B, S, DC), jnp.bfloat16),
            pltpu.VMEM((B, S, DC), jnp.bfloat16),
            pltpu.VMEM((DC, D), jnp.bfloat16),
            pltpu.VMEM((DC, D), jnp.bfloat16),
            pltpu.VMEM((DC, D), jnp.bfloat16),
            pltpu.VMEM((DC, D), jnp.bfloat16),
            pltpu.VMEM((2, CK, D), jnp.float32),
            pltpu.VMEM((2, CK, D), jnp.float32),
            pltpu.VMEM((D, Dr), jnp.float32),
            pltpu.SemaphoreType.DMA((3,)),
            pltpu.SemaphoreType.DMA((3,)),
            pltpu.SemaphoreType.DMA((5,)),
        ],
        compiler_params=_CompilerParams(collective_id=0),
    )(x_bf, Wdkv, Wuk, Wuv, Wq, Wqr, Wkr, Wo)
